# baseline (speedup 1.0000x reference)
import functools

import jax
import jax.numpy as jnp
from jax import lax
from jax.experimental import pallas as pl
from jax.experimental.pallas import tpu as pltpu
from jax.experimental.pallas import tpu_sc as plsc

N = 10000
E = 320000
D = 128
DO = 64
NP = 10240
NC = 2
NS = 16
NW = NC * NS
EPT = E // NW
CA = 100
NCA = EPT // CA
NBUF = 2
G = 10
NG = NCA // G
ZPT = NP // NS

_mesh = plsc.VectorSubcoreMesh(core_axis_name="c", subcore_axis_name="s")


@functools.partial(
    pl.kernel,
    out_type=jax.ShapeDtypeStruct((NC, NP), jnp.float32),
    mesh=_mesh,
    scratch_types=[
        pltpu.VMEM((NG, G, CA), jnp.int32),
        pltpu.VMEM((112,), jnp.float32),
        pltpu.VMEM_SHARED((NP,), jnp.float32),
        pltpu.SemaphoreType.DMA,
    ],
)
def _deg_kernel(er_hbm, zn_hbm, out_hbm, idx_v, ones_v, acc, dsem):
    c = lax.axis_index("c")
    s = lax.axis_index("s")
    wid = s * NC + c
    pltpu.sync_copy(zn_hbm.at[pl.ds(s * ZPT, ZPT)], acc.at[pl.ds(s * ZPT, ZPT)])
    pltpu.sync_copy(er_hbm.at[1, wid], idx_v)
    for i in range(7):
        ones_v[pl.ds(i * 16, 16)] = jnp.ones((16,), jnp.float32)
    plsc.subcore_barrier()

    def fire(j, carry):
        pltpu.async_copy(ones_v.at[pl.ds(0, CA)],
                         acc.at[idx_v.at[j // G, j % G]], dsem, add=True)
        return carry

    lax.fori_loop(0, NCA, fire, 0)

    def drain(j, carry):
        pltpu.make_async_copy(ones_v.at[pl.ds(0, CA)],
                              acc.at[idx_v.at[0, 0]], dsem).wait()
        return carry

    lax.fori_loop(0, NCA, drain, 0)
    plsc.subcore_barrier()
    pltpu.sync_copy(acc.at[pl.ds(s * ZPT, ZPT)], out_hbm.at[c, pl.ds(s * ZPT, ZPT)])


@functools.partial(
    pl.kernel,
    out_type=jax.ShapeDtypeStruct((NC, NP, D), jnp.float32),
    mesh=_mesh,
    scratch_types=(
        [pltpu.VMEM((G, CA), jnp.int32) for _ in range(4)]
        + [pltpu.VMEM((CA, D), jnp.float32) for _ in range(NBUF)]
        + [pltpu.VMEM_SHARED((NP, D), jnp.float32)]
        + [pltpu.SemaphoreType.DMA for _ in range(2 * NBUF + 1)]
    ),
)
def _agg_kernel(h2_hbm, er_hbm, z_hbm, out_hbm, *scr):
    sg0, sg1, dg0, dg1 = scr[0:4]
    sgb = (sg0, sg1)
    dgb = (dg0, dg1)
    rows = scr[4:4 + NBUF]
    acc = scr[4 + NBUF]
    gsems = scr[5 + NBUF:5 + 2 * NBUF]
    ssems = scr[5 + 2 * NBUF:5 + 3 * NBUF]
    isem = scr[5 + 3 * NBUF]
    c = lax.axis_index("c")
    s = lax.axis_index("s")
    wid = s * NC + c

    @pl.when(jnp.logical_and(c == 0, s < NS - 1))
    def _():
        pltpu.sync_copy(h2_hbm.at[pl.ds(s * ZPT, ZPT)], acc.at[pl.ds(s * ZPT, ZPT)])

    @pl.when(jnp.logical_and(c == 0, s == NS - 1))
    def _():
        pltpu.sync_copy(h2_hbm.at[pl.ds((NS - 1) * ZPT, N - (NS - 1) * ZPT)],
                        acc.at[pl.ds((NS - 1) * ZPT, N - (NS - 1) * ZPT)])

    @pl.when(c != 0)
    def _():
        pltpu.sync_copy(z_hbm.at[pl.ds(s * ZPT, ZPT)], acc.at[pl.ds(s * ZPT, ZPT)])

    def prefetch_group(g_idx, p):
        pltpu.async_copy(er_hbm.at[0, wid, g_idx], sgb[p], isem)
        pltpu.async_copy(er_hbm.at[1, wid, g_idx], dgb[p], isem)

    def wait_group(p):
        pltpu.make_async_copy(er_hbm.at[0, wid, 0], sgb[p], isem).wait()
        pltpu.make_async_copy(er_hbm.at[1, wid, 0], dgb[p], isem).wait()

    def start_gather(p, b_local, rb):
        pltpu.async_copy(h2_hbm.at[sgb[p].at[b_local]], rows[rb], gsems[rb])

    def wait_gather(rb):
        pltpu.make_async_copy(h2_hbm.at[sg0.at[0]], rows[rb], gsems[rb]).wait()

    def start_scatter(p, b_local, rb):
        pltpu.async_copy(rows[rb], acc.at[dgb[p].at[b_local]], ssems[rb],
                         add=True)

    def wait_scatter(rb):
        pltpu.make_async_copy(rows[rb], acc.at[dg0.at[0]], ssems[rb]).wait()

    pltpu.sync_copy(er_hbm.at[0, wid, 0], sg0)
    pltpu.sync_copy(er_hbm.at[1, wid, 0], dg0)
    prefetch_group(1, 1)
    plsc.subcore_barrier()

    for b in range(NBUF):
        start_gather(0, b, b)

    def pair(q, carry):
        for p in range(2):
            g = q * 2 + p

            @pl.when(jnp.logical_and(g >= 1, g + 1 < NG))
            def _():
                prefetch_group(g + 1, (p + 1) % 2)

            for b_local in range(G):
                rb = b_local % NBUF
                if b_local == G - NBUF + 1:
                    @pl.when(g + 1 < NG)
                    def _():
                        wait_group((p + 1) % 2)
                wait_gather(rb)
                start_scatter(p, b_local, rb)

                pb = (b_local - 1) % NBUF
                nxt = b_local + NBUF - 1

                def _regather():
                    wait_scatter(pb)
                    if nxt < G:
                        start_gather(p, nxt, pb)
                    else:
                        start_gather((p + 1) % 2, nxt - G, pb)

                if b_local == 0:
                    if nxt < G:
                        @pl.when(g > 0)
                        def _():
                            _regather()
                elif nxt + G * (NG - 1) >= NCA:
                    @pl.when(g < NG - 1)
                    def _():
                        _regather()
                else:
                    _regather()
        return carry

    lax.fori_loop(0, NG // 2, pair, 0)
    for b in range(NBUF):
        wait_scatter((NCA - NBUF + b) % NBUF)
    plsc.subcore_barrier()
    pltpu.sync_copy(acc.at[pl.ds(s * ZPT, ZPT)], out_hbm.at[c, pl.ds(s * ZPT, ZPT)])


def _mm_scale(x_ref, w_ref, deg_ref, h2_ref, dinv_ref):
    h = jnp.dot(x_ref[...], w_ref[...], preferred_element_type=jnp.float32)
    dsum = deg_ref[0] + deg_ref[1] + 1.0
    dinv = lax.rsqrt(dsum).reshape(NP, 1)[:N]
    dinv_ref[...] = dinv
    h2_ref[...] = h * dinv


def _final(parts_ref, dinv_ref, b1_ref, g1_ref, be1_ref,
           wr1_ref, br1_ref, wr2_ref, br2_ref, o_ref):
    inner = parts_ref[0] + parts_ref[1]
    g = inner[:N] * dinv_ref[...] + b1_ref[...]
    mean = jnp.mean(g, axis=0)
    var = jnp.mean((g - mean) ** 2, axis=0)
    xn = (g - mean) * lax.rsqrt(var + 1e-5)
    h = jnp.tanh(xn * g1_ref[...] + be1_ref[...])
    r = jnp.tanh(jnp.dot(h, wr1_ref[...], preferred_element_type=jnp.float32)
                 + br1_ref[...])
    o_ref[...] = (jnp.dot(r, wr2_ref[...], preferred_element_type=jnp.float32)
                  + br2_ref[...])


def kernel(x, edge_index, W1, b1, gamma1, beta1, Wr1, br1, Wr2, br2):
    er = edge_index.reshape(2, NW, NG, G, CA)
    zn = jnp.zeros((NP,), jnp.float32)
    znd = jnp.zeros((NP, D), jnp.float32)

    deg2 = _deg_kernel(er, zn)
    h2, dinv = pl.pallas_call(
        _mm_scale,
        out_shape=(jax.ShapeDtypeStruct((N, D), jnp.float32),
                   jax.ShapeDtypeStruct((N, 1), jnp.float32)))(x, W1, deg2)
    parts = _agg_kernel(h2, er, znd)
    out = pl.pallas_call(
        _final, out_shape=jax.ShapeDtypeStruct((N, DO), jnp.float32))(
            parts, dinv, b1, gamma1, beta1, Wr1, br1, Wr2, br2)
    return out


# speedup vs baseline: 1.1902x; 1.1902x over previous
import functools

import jax
import jax.numpy as jnp
from jax import lax
from jax.experimental import pallas as pl
from jax.experimental.pallas import tpu as pltpu
from jax.experimental.pallas import tpu_sc as plsc

N = 10000
E = 320000
D = 128
DO = 64
NP = 10240
NC = 2
NS = 16
NW = NC * NS
EPT = E // NW
CA = 50
NCA = EPT // CA
NBUF = 4
G = 20
NG = NCA // G
ZPT = NP // NS

_mesh = plsc.VectorSubcoreMesh(core_axis_name="c", subcore_axis_name="s")


@functools.partial(
    pl.kernel,
    out_type=jax.ShapeDtypeStruct((NC, NP), jnp.float32),
    mesh=_mesh,
    scratch_types=[
        pltpu.VMEM((NG, G, CA), jnp.int32),
        pltpu.VMEM((112,), jnp.float32),
        pltpu.VMEM_SHARED((NP,), jnp.float32),
        pltpu.SemaphoreType.DMA,
    ],
)
def _deg_kernel(er_hbm, zn_hbm, out_hbm, idx_v, ones_v, acc, dsem):
    c = lax.axis_index("c")
    s = lax.axis_index("s")
    wid = s * NC + c
    pltpu.sync_copy(zn_hbm.at[pl.ds(s * ZPT, ZPT)], acc.at[pl.ds(s * ZPT, ZPT)])
    pltpu.sync_copy(er_hbm.at[1, wid], idx_v)
    for i in range(7):
        ones_v[pl.ds(i * 16, 16)] = jnp.ones((16,), jnp.float32)
    plsc.subcore_barrier()

    def fire(j, carry):
        pltpu.async_copy(ones_v.at[pl.ds(0, CA)],
                         acc.at[idx_v.at[j // G, j % G]], dsem, add=True)
        return carry

    lax.fori_loop(0, NCA, fire, 0)

    def drain(j, carry):
        pltpu.make_async_copy(ones_v.at[pl.ds(0, CA)],
                              acc.at[idx_v.at[0, 0]], dsem).wait()
        return carry

    lax.fori_loop(0, NCA, drain, 0)
    plsc.subcore_barrier()
    pltpu.sync_copy(acc.at[pl.ds(s * ZPT, ZPT)], out_hbm.at[c, pl.ds(s * ZPT, ZPT)])


@functools.partial(
    pl.kernel,
    out_type=jax.ShapeDtypeStruct((NC, NP, D), jnp.float32),
    mesh=_mesh,
    scratch_types=(
        [pltpu.VMEM((G, CA), jnp.int32) for _ in range(4)]
        + [pltpu.VMEM((CA, D), jnp.float32) for _ in range(NBUF)]
        + [pltpu.VMEM_SHARED((NP, D), jnp.float32)]
        + [pltpu.SemaphoreType.DMA for _ in range(2 * NBUF + 1)]
    ),
)
def _agg_kernel(h2_hbm, er_hbm, z_hbm, out_hbm, *scr):
    sg0, sg1, dg0, dg1 = scr[0:4]
    sgb = (sg0, sg1)
    dgb = (dg0, dg1)
    rows = scr[4:4 + NBUF]
    acc = scr[4 + NBUF]
    gsems = scr[5 + NBUF:5 + 2 * NBUF]
    ssems = scr[5 + 2 * NBUF:5 + 3 * NBUF]
    isem = scr[5 + 3 * NBUF]
    c = lax.axis_index("c")
    s = lax.axis_index("s")
    wid = s * NC + c

    @pl.when(jnp.logical_and(c == 0, s < NS - 1))
    def _():
        pltpu.sync_copy(h2_hbm.at[pl.ds(s * ZPT, ZPT)], acc.at[pl.ds(s * ZPT, ZPT)])

    @pl.when(jnp.logical_and(c == 0, s == NS - 1))
    def _():
        pltpu.sync_copy(h2_hbm.at[pl.ds((NS - 1) * ZPT, N - (NS - 1) * ZPT)],
                        acc.at[pl.ds((NS - 1) * ZPT, N - (NS - 1) * ZPT)])

    @pl.when(c != 0)
    def _():
        pltpu.sync_copy(z_hbm.at[pl.ds(s * ZPT, ZPT)], acc.at[pl.ds(s * ZPT, ZPT)])

    def prefetch_group(g_idx, p):
        pltpu.async_copy(er_hbm.at[0, wid, g_idx], sgb[p], isem)
        pltpu.async_copy(er_hbm.at[1, wid, g_idx], dgb[p], isem)

    def wait_group(p):
        pltpu.make_async_copy(er_hbm.at[0, wid, 0], sgb[p], isem).wait()
        pltpu.make_async_copy(er_hbm.at[1, wid, 0], dgb[p], isem).wait()

    def start_gather(p, b_local, rb):
        pltpu.async_copy(h2_hbm.at[sgb[p].at[b_local]], rows[rb], gsems[rb])

    def wait_gather(rb):
        pltpu.make_async_copy(h2_hbm.at[sg0.at[0]], rows[rb], gsems[rb]).wait()

    def start_scatter(p, b_local, rb):
        pltpu.async_copy(rows[rb], acc.at[dgb[p].at[b_local]], ssems[rb],
                         add=True)

    def wait_scatter(rb):
        pltpu.make_async_copy(rows[rb], acc.at[dg0.at[0]], ssems[rb]).wait()

    pltpu.sync_copy(er_hbm.at[0, wid, 0], sg0)
    pltpu.sync_copy(er_hbm.at[1, wid, 0], dg0)
    prefetch_group(1, 1)
    plsc.subcore_barrier()

    for b in range(NBUF):
        start_gather(0, b, b)

    def pair(q, carry):
        for p in range(2):
            g = q * 2 + p

            @pl.when(jnp.logical_and(g >= 1, g + 1 < NG))
            def _():
                prefetch_group(g + 1, (p + 1) % 2)

            for b_local in range(G):
                rb = b_local % NBUF
                if b_local == G - NBUF + 1:
                    @pl.when(g + 1 < NG)
                    def _():
                        wait_group((p + 1) % 2)
                wait_gather(rb)
                start_scatter(p, b_local, rb)

                pb = (b_local - 1) % NBUF
                nxt = b_local + NBUF - 1

                def _regather():
                    wait_scatter(pb)
                    if nxt < G:
                        start_gather(p, nxt, pb)
                    else:
                        start_gather((p + 1) % 2, nxt - G, pb)

                if b_local == 0:
                    if nxt < G:
                        @pl.when(g > 0)
                        def _():
                            _regather()
                elif nxt + G * (NG - 1) >= NCA:
                    @pl.when(g < NG - 1)
                    def _():
                        _regather()
                else:
                    _regather()
        return carry

    lax.fori_loop(0, NG // 2, pair, 0)
    for b in range(NBUF):
        wait_scatter((NCA - NBUF + b) % NBUF)
    plsc.subcore_barrier()
    pltpu.sync_copy(acc.at[pl.ds(s * ZPT, ZPT)], out_hbm.at[c, pl.ds(s * ZPT, ZPT)])


def _mm_scale(x_ref, w_ref, deg_ref, h2_ref, dinv_ref):
    h = jnp.dot(x_ref[...], w_ref[...], preferred_element_type=jnp.float32)
    dsum = deg_ref[0] + deg_ref[1] + 1.0
    dinv = lax.rsqrt(dsum).reshape(NP, 1)[:N]
    dinv_ref[...] = dinv
    h2_ref[...] = h * dinv


def _final(parts_ref, dinv_ref, b1_ref, g1_ref, be1_ref,
           wr1_ref, br1_ref, wr2_ref, br2_ref, o_ref):
    inner = parts_ref[0] + parts_ref[1]
    g = inner[:N] * dinv_ref[...] + b1_ref[...]
    mean = jnp.mean(g, axis=0)
    var = jnp.mean((g - mean) ** 2, axis=0)
    xn = (g - mean) * lax.rsqrt(var + 1e-5)
    h = jnp.tanh(xn * g1_ref[...] + be1_ref[...])
    r = jnp.tanh(jnp.dot(h, wr1_ref[...], preferred_element_type=jnp.float32)
                 + br1_ref[...])
    o_ref[...] = (jnp.dot(r, wr2_ref[...], preferred_element_type=jnp.float32)
                  + br2_ref[...])


def kernel(x, edge_index, W1, b1, gamma1, beta1, Wr1, br1, Wr2, br2):
    er = edge_index.reshape(2, NW, NG, G, CA)
    zn = jnp.zeros((NP,), jnp.float32)
    znd = jnp.zeros((NP, D), jnp.float32)

    deg2 = _deg_kernel(er, zn)
    h2, dinv = pl.pallas_call(
        _mm_scale,
        out_shape=(jax.ShapeDtypeStruct((N, D), jnp.float32),
                   jax.ShapeDtypeStruct((N, 1), jnp.float32)))(x, W1, deg2)
    parts = _agg_kernel(h2, er, znd)
    out = pl.pallas_call(
        _final, out_shape=jax.ShapeDtypeStruct((N, DO), jnp.float32))(
            parts, dinv, b1, gamma1, beta1, Wr1, br1, Wr2, br2)
    return out
